# Initial kernel scaffold; baseline (speedup 1.0000x reference)
import jax
import jax.numpy as jnp
from jax.experimental import pallas as pl


def kernel(word_ids, med_input_ids, triage_input_ids, lab_input_ids, admin_input_ids, admin_ext_input_ids, scan1_input_ids, scan2_input_ids, scan3_input_ids, scan4_input_ids, indicator_input_ids, gcs_input_ids, seg_ids, age_ids, posi_ids, word_table, med_table, triage_table, lab_table, admin_table, admin_ext_table, scan1_table, scan2_table, scan3_table, scan4_table, indicator_table, gcs_table, seg_table, age_table, posi_table, gamma, beta):
    raise NotImplementedError("write your pallas kernel here")


# speedup vs baseline: 2.1784x; 2.1784x over previous
import functools

import jax
import jax.numpy as jnp
from jax import lax
from jax.experimental import pallas as pl
from jax.experimental.pallas import tpu as pltpu
from jax.experimental.pallas import tpu_sc as plsc

B, L, H = 1024, 200, 64
N = B * L
NW = 32
TOK_PER_W = N // NW
TPC = 8
SLOTS = 16
GROUPS_PER_W = TOK_PER_W // TPC
NGROUPS = N // TPC


def _lane_sum(v, perms):
    for p in perms:
        v = v + jnp.take_along_axis(v, p, axis=0)
    return v


def _rsqrt(x):
    i = lax.bitcast_convert_type(x, jnp.int32)
    y = lax.bitcast_convert_type(jnp.int32(0x5F3759DF) - (i >> 1), jnp.float32)
    for _ in range(3):
        y = y * (1.5 - 0.5 * x * y * y)
    return y


_mesh = plsc.VectorSubcoreMesh(core_axis_name="c", subcore_axis_name="s")


@functools.partial(
    pl.kernel,
    mesh=_mesh,
    compiler_params=pltpu.CompilerParams(use_tc_tiling_on_sc=False),
    out_type=jax.ShapeDtypeStruct((N, H), jnp.float32),
    scratch_types=[
        pltpu.VMEM((TPC * SLOTS,), jnp.int32),
        pltpu.VMEM((TPC * SLOTS, H), jnp.float32),
        pltpu.VMEM((TPC, H), jnp.float32),
        pltpu.VMEM((H,), jnp.float32),
        pltpu.VMEM((H,), jnp.float32),
        pltpu.SemaphoreType.DMA,
    ],
)
def _sc_embed_ln(ids_ref, table_ref, gamma_ref, beta_ref, out_ref,
                 idx_v, rows_v, out_v, gamma_v, beta_v, sem):
    wid = lax.axis_index("s") * 2 + lax.axis_index("c")
    pltpu.sync_copy(gamma_ref, gamma_v)
    pltpu.sync_copy(beta_ref, beta_v)
    gvec = [gamma_v[pl.ds(16 * w, 16)] for w in range(4)]
    bvec = [beta_v[pl.ds(16 * w, 16)] for w in range(4)]
    iota = lax.iota(jnp.int32, 16)
    perms = [jnp.bitwise_and(iota + s, 15) for s in (8, 4, 2, 1)]
    base_g = wid * GROUPS_PER_W

    def body(g, carry):
        grp = base_g + g
        pltpu.sync_copy(ids_ref.at[grp], idx_v)
        pltpu.async_copy(table_ref.at[idx_v], rows_v, sem).wait()
        for t in range(TPC):
            accs = []
            for w in range(4):
                a = rows_v[t * SLOTS, pl.ds(16 * w, 16)]
                for f in range(1, 15):
                    a = a + rows_v[t * SLOTS + f, pl.ds(16 * w, 16)]
                accs.append(a)
            s = (accs[0] + accs[1]) + (accs[2] + accs[3])
            q = (accs[0] * accs[0] + accs[1] * accs[1]) + \
                (accs[2] * accs[2] + accs[3] * accs[3])
            s = _lane_sum(s, perms)
            q = _lane_sum(q, perms)
            mu = s * (1.0 / 64.0)
            var = q * (1.0 / 64.0) - mu * mu
            r = _rsqrt(var + 1e-12)
            for w in range(4):
                out_v[t, pl.ds(16 * w, 16)] = \
                    (accs[w] - mu) * r * gvec[w] + bvec[w]
        pltpu.sync_copy(out_v, out_ref.at[pl.ds(grp * TPC, TPC)])
        return carry

    lax.fori_loop(0, GROUPS_PER_W, body, 0)


def kernel(word_ids, med_input_ids, triage_input_ids, lab_input_ids,
           admin_input_ids, admin_ext_input_ids, scan1_input_ids,
           scan2_input_ids, scan3_input_ids, scan4_input_ids,
           indicator_input_ids, gcs_input_ids, seg_ids, age_ids, posi_ids,
           word_table, med_table, triage_table, lab_table, admin_table,
           admin_ext_table, scan1_table, scan2_table, scan3_table,
           scan4_table, indicator_table, gcs_table, seg_table, age_table,
           posi_table, gamma, beta):
    ids = [word_ids, med_input_ids, triage_input_ids, lab_input_ids,
           admin_input_ids, admin_ext_input_ids, scan1_input_ids,
           scan2_input_ids, scan3_input_ids, scan4_input_ids,
           indicator_input_ids, gcs_input_ids, seg_ids, age_ids, posi_ids]
    tables = [word_table, med_table, triage_table, lab_table, admin_table,
              admin_ext_table, scan1_table, scan2_table, scan3_table,
              scan4_table, indicator_table, gcs_table, seg_table, age_table,
              posi_table]
    big = jnp.concatenate(
        tables + [jnp.zeros((1, H), jnp.float32)], axis=0)
    offs, o = [], 0
    for t in tables:
        offs.append(o)
        o += t.shape[0]
    dummy = o
    cols = [i.reshape(N).astype(jnp.int32) + jnp.int32(off)
            for i, off in zip(ids, offs)]
    cols.append(jnp.full((N,), dummy, jnp.int32))
    idx = jnp.stack(cols, axis=1).reshape(NGROUPS, TPC * SLOTS)
    out = _sc_embed_ln(idx, big, gamma, beta)
    return out.reshape(B, L, H)
